# Initial kernel scaffold; baseline (speedup 1.0000x reference)
import jax
import jax.numpy as jnp
from jax.experimental import pallas as pl
from jax.experimental.pallas import tpu as pltpu

D_MODEL = 768
N_LAYER = 4
VOCAB = 50257
D_STATE = 16
D_CONV = 4
D_INNER = 2 * D_MODEL
DT_RANK = 48
L = 2048

CHUNK = 256
N_CHUNK = L // CHUNK
GROWS = 256
BV = 512
NV = (VOCAB + BV - 1) // BV


def _gather_kernel(cidx_ref, e3_ref, out_ref, sem):
    i = pl.program_id(0)
    base = i * GROWS
    for r in range(GROWS):
        tok = cidx_ref[base + r]
        pltpu.make_async_copy(e3_ref.at[tok], out_ref.at[r], sem).start()
    for r in range(GROWS):
        tok = cidx_ref[base + r]
        pltpu.make_async_copy(e3_ref.at[tok], out_ref.at[r], sem).wait()


def _gather_call(cidx, e3):
    n = cidx.shape[0]
    return pl.pallas_call(
        _gather_kernel,
        out_shape=jax.ShapeDtypeStruct((n, 6, 128), jnp.float32),
        grid_spec=pltpu.PrefetchScalarGridSpec(
            num_scalar_prefetch=1,
            grid=(n // GROWS,),
            in_specs=[pl.BlockSpec(memory_space=pltpu.ANY)],
            out_specs=pl.BlockSpec((GROWS, 6, 128), lambda i, sref: (i, 0, 0)),
        ),
        scratch_shapes=[pltpu.SemaphoreType.DMA],
        compiler_params=pltpu.CompilerParams(
            dimension_semantics=("arbitrary",),
            vmem_limit_bytes=32 * 1024 * 1024,
        ),
        name="embed_gather",
    )(cidx, e3)


def _layer_kernel(x_ref, inw_ref, cwt_ref, cb_ref, xpw_ref, dtw_ref, dtb_ref,
                  alt_ref, dp_ref, ow_ref, nw_ref, xo_ref,
                  h_s, tail_s, conv_s, dt_s, dtx_s, b_s, c_s, y_s):
    i = pl.program_id(0)
    x = x_ref[...]
    ms = jnp.mean(x * x, axis=1, keepdims=True)
    xn = x * jax.lax.rsqrt(ms + 1e-5) * nw_ref[...]
    xz = jnp.einsum("lk,nk->ln", xn, inw_ref[...],
                    preferred_element_type=jnp.float32)
    xi_raw = xz[:, :D_INNER]
    z = xz[:, D_INNER:]

    @pl.when(i == 0)
    def _():
        tail_s[...] = jnp.zeros_like(tail_s)
        h_s[...] = jnp.zeros_like(h_s)

    conv_s[pl.ds(5, 3), :] = tail_s[pl.ds(5, 3), :]
    conv_s[pl.ds(8, CHUNK), :] = xi_raw
    tail_s[...] = xi_raw[CHUNK - 8:CHUNK, :]

    cw = cwt_ref[...]
    conv = (cb_ref[...]
            + conv_s[pl.ds(5, CHUNK), :] * cw[0:1, :]
            + conv_s[pl.ds(6, CHUNK), :] * cw[1:2, :]
            + conv_s[pl.ds(7, CHUNK), :] * cw[2:3, :]
            + conv_s[pl.ds(8, CHUNK), :] * cw[3:4, :])
    xi = conv * jax.nn.sigmoid(conv)

    xdbl = jnp.einsum("lk,nk->ln", xi, xpw_ref[...],
                      preferred_element_type=jnp.float32)
    dt = jax.nn.softplus(
        jnp.einsum("lk,nk->ln", xdbl, dtw_ref[...],
                   preferred_element_type=jnp.float32) + dtb_ref[...])

    b_s[...] = xdbl[:, DT_RANK:DT_RANK + D_STATE]
    c_s[...] = xdbl[:, DT_RANK + D_STATE:DT_RANK + 2 * D_STATE]
    dt_s[...] = dt
    dtx_s[...] = dt * xi

    A = -jnp.exp(alt_ref[...])

    def tile_body(j, h):
        off = pl.multiple_of(j * 8, 8)
        dt8 = dt_s[pl.ds(off, 8), :]
        dtx8 = dtx_s[pl.ds(off, 8), :]
        b8t = b_s[pl.ds(off, 8), :].T
        c8t = c_s[pl.ds(off, 8), :].T
        rows = []
        for r in range(8):
            a = jnp.exp(dt8[r:r + 1, :] * A)
            h = a * h + dtx8[r:r + 1, :] * b8t[:, r:r + 1]
            rows.append(jnp.sum(h * c8t[:, r:r + 1], axis=0, keepdims=True))
        y_s[pl.ds(off, 8), :] = jnp.concatenate(rows, axis=0)
        return h

    hn = jax.lax.fori_loop(0, CHUNK // 8, tile_body, h_s[...])
    h_s[...] = hn

    y = (y_s[...] + xi * dp_ref[...]) * (z * jax.nn.sigmoid(z))
    xo_ref[...] = x + jnp.einsum("ld,md->lm", y, ow_ref[...],
                                 preferred_element_type=jnp.float32)


def _layer_call(x, inw, cwt, cb, xpw, dtw, dtb, alt, dp, ow, nw):
    full = lambda shape: pl.BlockSpec(shape, lambda i: tuple(0 for _ in shape))
    return pl.pallas_call(
        _layer_kernel,
        out_shape=jax.ShapeDtypeStruct((L, D_MODEL), jnp.float32),
        grid=(N_CHUNK,),
        in_specs=[
            pl.BlockSpec((CHUNK, D_MODEL), lambda i: (i, 0)),
            full((2 * D_INNER, D_MODEL)),
            full((D_CONV, D_INNER)),
            full((1, D_INNER)),
            full((128, D_INNER)),
            full((D_INNER, 128)),
            full((1, D_INNER)),
            full((D_STATE, D_INNER)),
            full((1, D_INNER)),
            full((D_MODEL, D_INNER)),
            full((1, D_MODEL)),
        ],
        out_specs=pl.BlockSpec((CHUNK, D_MODEL), lambda i: (i, 0)),
        scratch_shapes=[
            pltpu.VMEM((D_STATE, D_INNER), jnp.float32),
            pltpu.VMEM((8, D_INNER), jnp.float32),
            pltpu.VMEM((CHUNK + 8, D_INNER), jnp.float32),
            pltpu.VMEM((CHUNK, D_INNER), jnp.float32),
            pltpu.VMEM((CHUNK, D_INNER), jnp.float32),
            pltpu.VMEM((CHUNK, D_STATE), jnp.float32),
            pltpu.VMEM((CHUNK, D_STATE), jnp.float32),
            pltpu.VMEM((CHUNK, D_INNER), jnp.float32),
        ],
        compiler_params=pltpu.CompilerParams(
            dimension_semantics=("arbitrary",),
            vmem_limit_bytes=60 * 1024 * 1024,
        ),
        name="mamba_layer",
    )(x, inw, cwt, cb, xpw, dtw, dtb, alt, dp, ow, nw)


def _head_kernel(x4_ref, etgt_ref, nfw_ref, e_ref, out_ref, lse_ref, tdot_ref,
                 xf_s, s_acc):
    j = pl.program_id(0)

    @pl.when(j == 0)
    def _():
        x4 = x4_ref[...]
        xf = x4 * jax.lax.rsqrt(jnp.mean(x4 * x4, axis=1, keepdims=True)
                                + 1e-5) * nfw_ref[...]
        xf_s[...] = xf
        tdot_ref[...] = jnp.sum(xf * etgt_ref[...], axis=1, keepdims=True)
        s_acc[...] = jnp.zeros_like(s_acc)

    xf = xf_s[...]
    blk = jnp.einsum("ld,vd->lv", xf, e_ref[...],
                     preferred_element_type=jnp.float32)
    out_ref[...] = blk
    cols = j * BV + jax.lax.broadcasted_iota(jnp.int32, (1, BV), 1)
    blkm = jnp.where(cols < VOCAB, blk, -jnp.inf)
    s_acc[...] = s_acc[...] + jnp.sum(jnp.exp(blkm), axis=1, keepdims=True)

    @pl.when(j == NV - 1)
    def _():
        lse_ref[...] = jnp.log(s_acc[...])


def _head_call(x4, etgt, nfw, emb):
    full = lambda shape: pl.BlockSpec(shape, lambda j: tuple(0 for _ in shape))
    return pl.pallas_call(
        _head_kernel,
        out_shape=[
            jax.ShapeDtypeStruct((L, VOCAB), jnp.float32),
            jax.ShapeDtypeStruct((L, 1), jnp.float32),
            jax.ShapeDtypeStruct((L, 1), jnp.float32),
        ],
        grid=(NV,),
        in_specs=[
            full((L, D_MODEL)),
            full((L, D_MODEL)),
            full((1, D_MODEL)),
            pl.BlockSpec((BV, D_MODEL), lambda j: (j, 0)),
        ],
        out_specs=[
            pl.BlockSpec((L, BV), lambda j: (0, j)),
            full((L, 1)),
            full((L, 1)),
        ],
        scratch_shapes=[
            pltpu.VMEM((L, D_MODEL), jnp.float32),
            pltpu.VMEM((L, 1), jnp.float32),
        ],
        compiler_params=pltpu.CompilerParams(
            dimension_semantics=("arbitrary",),
            vmem_limit_bytes=60 * 1024 * 1024,
        ),
        name="lm_head",
    )(x4, etgt, nfw, emb)


def kernel(idx, targets, embedding, norm_w, in_proj_w, conv_w, conv_b,
           x_proj_w, dt_proj_w, dt_proj_b, A_log, D_param, out_proj_w,
           norm_f_w):
    tclip = jnp.clip(targets[0], 0, VOCAB - 1)
    cidx = jnp.concatenate([idx[0], tclip])
    e3 = embedding.reshape(VOCAB, 6, 128)
    gathered = _gather_call(cidx, e3)
    x = gathered[:L].reshape(L, D_MODEL)
    etgt = gathered[L:].reshape(L, D_MODEL)

    for l in range(N_LAYER):
        xpw = jnp.pad(x_proj_w[l], ((0, 128 - (DT_RANK + 2 * D_STATE)), (0, 0)))
        dtw = jnp.pad(dt_proj_w[l], ((0, 0), (0, 128 - DT_RANK)))
        x = _layer_call(
            x,
            in_proj_w[l],
            conv_w[l].T,
            conv_b[l][None, :],
            xpw,
            dtw,
            dt_proj_b[l][None, :],
            A_log[l].T,
            D_param[l][None, :],
            out_proj_w[l],
            norm_w[l][None, :],
        )

    logits2d, lse, tdot = _head_call(x, etgt, norm_f_w[None, :], embedding)
    logits = logits2d.reshape(1, L, VOCAB)

    maskv = (targets[0] >= 0).astype(jnp.float32)
    nll = (lse[:, 0] - tdot[:, 0]) * maskv
    loss = jnp.sum(nll) / jnp.maximum(jnp.sum(maskv), 1.0)
    return (logits, loss)


# speedup vs baseline: 14.7170x; 14.7170x over previous
import jax
import jax.numpy as jnp
from jax.experimental import pallas as pl
from jax.experimental.pallas import tpu as pltpu

D_MODEL = 768
N_LAYER = 4
VOCAB = 50257
D_STATE = 16
D_CONV = 4
D_INNER = 2 * D_MODEL
DT_RANK = 48
L = 2048

CHUNK = 256
N_CHUNK = L // CHUNK
GROWS = 256
BV = 512
NV = (VOCAB + BV - 1) // BV


def _gather_kernel(cidx_ref, e3_ref, out_ref, sem):
    i = pl.program_id(0)
    base = i * GROWS
    for r in range(GROWS):
        tok = cidx_ref[base + r]
        pltpu.make_async_copy(e3_ref.at[tok], out_ref.at[r], sem).start()
    for r in range(GROWS):
        tok = cidx_ref[base + r]
        pltpu.make_async_copy(e3_ref.at[tok], out_ref.at[r], sem).wait()


def _gather_call(cidx, e3):
    n = cidx.shape[0]
    return pl.pallas_call(
        _gather_kernel,
        out_shape=jax.ShapeDtypeStruct((n, 6, 128), jnp.float32),
        grid_spec=pltpu.PrefetchScalarGridSpec(
            num_scalar_prefetch=1,
            grid=(n // GROWS,),
            in_specs=[pl.BlockSpec(memory_space=pl.ANY)],
            out_specs=pl.BlockSpec((GROWS, 6, 128), lambda i, sref: (i, 0, 0)),
            scratch_shapes=[pltpu.SemaphoreType.DMA],
        ),
        compiler_params=pltpu.CompilerParams(
            dimension_semantics=("arbitrary",),
            vmem_limit_bytes=32 * 1024 * 1024,
        ),
        name="embed_gather",
    )(cidx, e3)


def _layer_kernel(x_ref, inw_ref, cwt_ref, cb_ref, xpw_ref, dtw_ref, dtb_ref,
                  alt_ref, dp_ref, ow_ref, nw_ref, xo_ref,
                  h_s, tail_s, conv_s, dt_s, dtx_s, b_s, c_s, y_s):
    i = pl.program_id(0)
    x = x_ref[...]
    ms = jnp.mean(x * x, axis=1, keepdims=True)
    xn = x * jax.lax.rsqrt(ms + 1e-5) * nw_ref[...]
    xz = jnp.einsum("lk,nk->ln", xn, inw_ref[...],
                    preferred_element_type=jnp.float32)
    xi_raw = xz[:, :D_INNER]
    z = xz[:, D_INNER:]

    @pl.when(i == 0)
    def _():
        tail_s[...] = jnp.zeros_like(tail_s)
        h_s[...] = jnp.zeros_like(h_s)

    conv_s[pl.ds(5, 3), :] = tail_s[pl.ds(5, 3), :]
    conv_s[pl.ds(8, CHUNK), :] = xi_raw
    tail_s[...] = xi_raw[CHUNK - 8:CHUNK, :]

    cw = cwt_ref[...]
    conv = (cb_ref[...]
            + conv_s[pl.ds(5, CHUNK), :] * cw[0:1, :]
            + conv_s[pl.ds(6, CHUNK), :] * cw[1:2, :]
            + conv_s[pl.ds(7, CHUNK), :] * cw[2:3, :]
            + conv_s[pl.ds(8, CHUNK), :] * cw[3:4, :])
    xi = conv * jax.nn.sigmoid(conv)

    xdbl = jnp.einsum("lk,nk->ln", xi, xpw_ref[...],
                      preferred_element_type=jnp.float32)
    dt = jax.nn.softplus(
        jnp.einsum("lk,nk->ln", xdbl, dtw_ref[...],
                   preferred_element_type=jnp.float32) + dtb_ref[...])

    b_s[...] = xdbl[:, DT_RANK:DT_RANK + D_STATE]
    c_s[...] = xdbl[:, DT_RANK + D_STATE:DT_RANK + 2 * D_STATE]
    dt_s[...] = dt
    dtx_s[...] = dt * xi

    A = -jnp.exp(alt_ref[...])

    def tile_body(j, h):
        off = pl.multiple_of(j * 8, 8)
        dt8 = dt_s[pl.ds(off, 8), :]
        dtx8 = dtx_s[pl.ds(off, 8), :]
        b8t = b_s[pl.ds(off, 8), :].T
        c8t = c_s[pl.ds(off, 8), :].T
        rows = []
        for r in range(8):
            a = jnp.exp(dt8[r:r + 1, :] * A)
            h = a * h + dtx8[r:r + 1, :] * b8t[:, r:r + 1]
            rows.append(jnp.sum(h * c8t[:, r:r + 1], axis=0, keepdims=True))
        y_s[pl.ds(off, 8), :] = jnp.concatenate(rows, axis=0)
        return h

    hn = jax.lax.fori_loop(0, CHUNK // 8, tile_body, h_s[...])
    h_s[...] = hn

    y = (y_s[...] + xi * dp_ref[...]) * (z * jax.nn.sigmoid(z))
    xo_ref[...] = x + jnp.einsum("ld,md->lm", y, ow_ref[...],
                                 preferred_element_type=jnp.float32)


def _layer_call(x, inw, cwt, cb, xpw, dtw, dtb, alt, dp, ow, nw):
    full = lambda shape: pl.BlockSpec(shape, lambda i: tuple(0 for _ in shape))
    return pl.pallas_call(
        _layer_kernel,
        out_shape=jax.ShapeDtypeStruct((L, D_MODEL), jnp.float32),
        grid=(N_CHUNK,),
        in_specs=[
            pl.BlockSpec((CHUNK, D_MODEL), lambda i: (i, 0)),
            full((2 * D_INNER, D_MODEL)),
            full((D_CONV, D_INNER)),
            full((1, D_INNER)),
            full((128, D_INNER)),
            full((D_INNER, 128)),
            full((1, D_INNER)),
            full((D_STATE, D_INNER)),
            full((1, D_INNER)),
            full((D_MODEL, D_INNER)),
            full((1, D_MODEL)),
        ],
        out_specs=pl.BlockSpec((CHUNK, D_MODEL), lambda i: (i, 0)),
        scratch_shapes=[
            pltpu.VMEM((D_STATE, D_INNER), jnp.float32),
            pltpu.VMEM((8, D_INNER), jnp.float32),
            pltpu.VMEM((CHUNK + 8, D_INNER), jnp.float32),
            pltpu.VMEM((CHUNK, D_INNER), jnp.float32),
            pltpu.VMEM((CHUNK, D_INNER), jnp.float32),
            pltpu.VMEM((CHUNK, D_STATE), jnp.float32),
            pltpu.VMEM((CHUNK, D_STATE), jnp.float32),
            pltpu.VMEM((CHUNK, D_INNER), jnp.float32),
        ],
        compiler_params=pltpu.CompilerParams(
            dimension_semantics=("arbitrary",),
            vmem_limit_bytes=60 * 1024 * 1024,
        ),
        name="mamba_layer",
    )(x, inw, cwt, cb, xpw, dtw, dtb, alt, dp, ow, nw)


def _head_kernel(x4_ref, etgt_ref, nfw_ref, e_ref, out_ref, lse_ref, tdot_ref,
                 xf_s, s_acc):
    j = pl.program_id(0)

    @pl.when(j == 0)
    def _():
        x4 = x4_ref[...]
        xf = x4 * jax.lax.rsqrt(jnp.mean(x4 * x4, axis=1, keepdims=True)
                                + 1e-5) * nfw_ref[...]
        xf_s[...] = xf
        tdot_ref[...] = jnp.sum(xf * etgt_ref[...], axis=1, keepdims=True)
        s_acc[...] = jnp.zeros_like(s_acc)

    xf = xf_s[...]
    blk = jnp.einsum("ld,vd->lv", xf, e_ref[...],
                     preferred_element_type=jnp.float32)
    out_ref[...] = blk
    cols = j * BV + jax.lax.broadcasted_iota(jnp.int32, (1, BV), 1)
    blkm = jnp.where(cols < VOCAB, blk, -jnp.inf)
    s_acc[...] = s_acc[...] + jnp.sum(jnp.exp(blkm), axis=1, keepdims=True)

    @pl.when(j == NV - 1)
    def _():
        lse_ref[...] = jnp.log(s_acc[...])


def _head_call(x4, etgt, nfw, emb):
    full = lambda shape: pl.BlockSpec(shape, lambda j: tuple(0 for _ in shape))
    return pl.pallas_call(
        _head_kernel,
        out_shape=[
            jax.ShapeDtypeStruct((L, VOCAB), jnp.float32),
            jax.ShapeDtypeStruct((L, 1), jnp.float32),
            jax.ShapeDtypeStruct((L, 1), jnp.float32),
        ],
        grid=(NV,),
        in_specs=[
            full((L, D_MODEL)),
            full((L, D_MODEL)),
            full((1, D_MODEL)),
            pl.BlockSpec((BV, D_MODEL), lambda j: (j, 0)),
        ],
        out_specs=[
            pl.BlockSpec((L, BV), lambda j: (0, j)),
            full((L, 1)),
            full((L, 1)),
        ],
        scratch_shapes=[
            pltpu.VMEM((L, D_MODEL), jnp.float32),
            pltpu.VMEM((L, 1), jnp.float32),
        ],
        compiler_params=pltpu.CompilerParams(
            dimension_semantics=("arbitrary",),
            vmem_limit_bytes=60 * 1024 * 1024,
        ),
        name="lm_head",
    )(x4, etgt, nfw, emb)


def kernel(idx, targets, embedding, norm_w, in_proj_w, conv_w, conv_b,
           x_proj_w, dt_proj_w, dt_proj_b, A_log, D_param, out_proj_w,
           norm_f_w):
    tclip = jnp.clip(targets[0], 0, VOCAB - 1)
    cidx = jnp.concatenate([idx[0], tclip])
    e3 = embedding.reshape(VOCAB, 6, 128)
    gathered = _gather_call(cidx, e3)
    x = gathered[:L].reshape(L, D_MODEL)
    etgt = gathered[L:].reshape(L, D_MODEL)

    for l in range(N_LAYER):
        xpw = jnp.pad(x_proj_w[l], ((0, 128 - (DT_RANK + 2 * D_STATE)), (0, 0)))
        dtw = jnp.pad(dt_proj_w[l], ((0, 0), (0, 128 - DT_RANK)))
        x = _layer_call(
            x,
            in_proj_w[l],
            conv_w[l].T,
            conv_b[l][None, :],
            xpw,
            dtw,
            dt_proj_b[l][None, :],
            A_log[l].T,
            D_param[l][None, :],
            out_proj_w[l],
            norm_w[l][None, :],
        )

    logits2d, lse, tdot = _head_call(x, etgt, norm_f_w[None, :], embedding)
    logits = logits2d.reshape(1, L, VOCAB)

    maskv = (targets[0] >= 0).astype(jnp.float32)
    nll = (lse[:, 0] - tdot[:, 0]) * maskv
    loss = jnp.sum(nll) / jnp.maximum(jnp.sum(maskv), 1.0)
    return (logits, loss)


# speedup vs baseline: 17.5258x; 1.1909x over previous
import jax
import jax.numpy as jnp
from jax.experimental import pallas as pl
from jax.experimental.pallas import tpu as pltpu

D_MODEL = 768
N_LAYER = 4
VOCAB = 50257
D_STATE = 16
D_CONV = 4
D_INNER = 2 * D_MODEL
DT_RANK = 48
L = 2048

CHUNK = 256
N_CHUNK = L // CHUNK
GROWS = 128
BV = 512
NV = (VOCAB + BV - 1) // BV
NVC = (NV + 1) // 2


def _gather_issue(cidx_ref, emb_ref, chunk_s, sems, step, slot):
    base = step * GROWS
    for r in range(GROWS):
        tok = cidx_ref[base + r]
        ck = pl.multiple_of((tok >> 3) << 3, 8)
        pltpu.make_async_copy(emb_ref.at[pl.ds(ck, 8), :],
                              chunk_s.at[slot, r], sems.at[slot]).start()


def _gather_kernel(cidx_ref, emb_ref, out_ref, chunk_s, sems):
    i = pl.program_id(0)
    ng = pl.num_programs(0)

    @pl.when(i == 0)
    def _():
        _gather_issue(cidx_ref, emb_ref, chunk_s, sems, 0, 0)

    @pl.when(i + 1 < ng)
    def _():
        _gather_issue(cidx_ref, emb_ref, chunk_s, sems, i + 1, (i + 1) % 2)

    slot = i % 2
    for r in range(GROWS):
        pltpu.make_async_copy(emb_ref.at[pl.ds(0, 8), :],
                              chunk_s.at[slot, r], sems.at[slot]).wait()
    for r in range(GROWS):
        t7 = cidx_ref[i * GROWS + r] & 7
        rolled = pltpu.roll(chunk_s[slot, r], -t7, axis=0)
        out_ref[r:r + 1, :] = rolled[0:1, :]


def _gather_call(cidx, emb):
    n = cidx.shape[0]
    return pl.pallas_call(
        _gather_kernel,
        out_shape=jax.ShapeDtypeStruct((n, D_MODEL), jnp.float32),
        grid_spec=pltpu.PrefetchScalarGridSpec(
            num_scalar_prefetch=1,
            grid=(n // GROWS,),
            in_specs=[pl.BlockSpec(memory_space=pl.ANY)],
            out_specs=pl.BlockSpec((GROWS, D_MODEL), lambda i, sref: (i, 0)),
            scratch_shapes=[
                pltpu.VMEM((2, GROWS, 8, D_MODEL), jnp.float32),
                pltpu.SemaphoreType.DMA((2,)),
            ],
        ),
        compiler_params=pltpu.CompilerParams(
            dimension_semantics=("arbitrary",),
            vmem_limit_bytes=32 * 1024 * 1024,
        ),
        name="embed_gather",
    )(cidx, emb)


def _layer_kernel(x_ref, inw_ref, cwt_ref, cb_ref, xpw_ref, dtw_ref, dtb_ref,
                  alt_ref, dp_ref, ow_ref, nw_ref, xo_ref,
                  h_s, tail_s, conv_s, dt_s, dtx_s, b_s, c_s, y_s):
    i = pl.program_id(0)
    x = x_ref[...]
    ms = jnp.mean(x * x, axis=1, keepdims=True)
    xn = x * jax.lax.rsqrt(ms + 1e-5) * nw_ref[...]
    xz = jnp.einsum("lk,nk->ln", xn, inw_ref[...],
                    preferred_element_type=jnp.float32)
    xi_raw = xz[:, :D_INNER]
    z = xz[:, D_INNER:]

    @pl.when(i == 0)
    def _():
        tail_s[...] = jnp.zeros_like(tail_s)
        h_s[...] = jnp.zeros_like(h_s)

    conv_s[pl.ds(5, 3), :] = tail_s[pl.ds(5, 3), :]
    conv_s[pl.ds(8, CHUNK), :] = xi_raw
    tail_s[...] = xi_raw[CHUNK - 8:CHUNK, :]

    cw = cwt_ref[...]
    conv = (cb_ref[...]
            + conv_s[pl.ds(5, CHUNK), :] * cw[0:1, :]
            + conv_s[pl.ds(6, CHUNK), :] * cw[1:2, :]
            + conv_s[pl.ds(7, CHUNK), :] * cw[2:3, :]
            + conv_s[pl.ds(8, CHUNK), :] * cw[3:4, :])
    xi = conv * jax.nn.sigmoid(conv)

    xdbl = jnp.einsum("lk,nk->ln", xi, xpw_ref[...],
                      preferred_element_type=jnp.float32)
    dt = jax.nn.softplus(
        jnp.einsum("lk,nk->ln", xdbl, dtw_ref[...],
                   preferred_element_type=jnp.float32) + dtb_ref[...])

    b_s[...] = xdbl[:, DT_RANK:DT_RANK + D_STATE]
    c_s[...] = xdbl[:, DT_RANK + D_STATE:DT_RANK + 2 * D_STATE]
    dt_s[...] = dt
    dtx_s[...] = dt * xi

    A = -jnp.exp(alt_ref[...])

    def tile_body(j, h):
        off = pl.multiple_of(j * 8, 8)
        dt8 = dt_s[pl.ds(off, 8), :]
        dtx8 = dtx_s[pl.ds(off, 8), :]
        b8t = b_s[pl.ds(off, 8), :].T
        c8t = c_s[pl.ds(off, 8), :].T
        rows = []
        for r in range(8):
            a = jnp.exp(dt8[r:r + 1, :] * A)
            h = a * h + dtx8[r:r + 1, :] * b8t[:, r:r + 1]
            rows.append(jnp.sum(h * c8t[:, r:r + 1], axis=0, keepdims=True))
        y_s[pl.ds(off, 8), :] = jnp.concatenate(rows, axis=0)
        return h

    hn = jax.lax.fori_loop(0, CHUNK // 8, tile_body, h_s[...])
    h_s[...] = hn

    y = (y_s[...] + xi * dp_ref[...]) * (z * jax.nn.sigmoid(z))
    xo_ref[...] = x + jnp.einsum("ld,md->lm", y, ow_ref[...],
                                 preferred_element_type=jnp.float32)


def _layer_call(x, inw, cwt, cb, xpw, dtw, dtb, alt, dp, ow, nw):
    full = lambda shape: pl.BlockSpec(shape, lambda i: tuple(0 for _ in shape))
    return pl.pallas_call(
        _layer_kernel,
        out_shape=jax.ShapeDtypeStruct((L, D_MODEL), jnp.float32),
        grid=(N_CHUNK,),
        in_specs=[
            pl.BlockSpec((CHUNK, D_MODEL), lambda i: (i, 0)),
            full((2 * D_INNER, D_MODEL)),
            full((D_CONV, D_INNER)),
            full((1, D_INNER)),
            full((128, D_INNER)),
            full((D_INNER, 128)),
            full((1, D_INNER)),
            full((D_STATE, D_INNER)),
            full((1, D_INNER)),
            full((D_MODEL, D_INNER)),
            full((1, D_MODEL)),
        ],
        out_specs=pl.BlockSpec((CHUNK, D_MODEL), lambda i: (i, 0)),
        scratch_shapes=[
            pltpu.VMEM((D_STATE, D_INNER), jnp.float32),
            pltpu.VMEM((8, D_INNER), jnp.float32),
            pltpu.VMEM((CHUNK + 8, D_INNER), jnp.float32),
            pltpu.VMEM((CHUNK, D_INNER), jnp.float32),
            pltpu.VMEM((CHUNK, D_INNER), jnp.float32),
            pltpu.VMEM((CHUNK, D_STATE), jnp.float32),
            pltpu.VMEM((CHUNK, D_STATE), jnp.float32),
            pltpu.VMEM((CHUNK, D_INNER), jnp.float32),
        ],
        compiler_params=pltpu.CompilerParams(
            dimension_semantics=("arbitrary",),
            vmem_limit_bytes=60 * 1024 * 1024,
        ),
        name="mamba_layer",
    )(x, inw, cwt, cb, xpw, dtw, dtb, alt, dp, ow, nw)


def _head_kernel(x4_ref, etgt_ref, nfw_ref, e_ref, out_ref, s_ref, tdot_ref,
                 xf_s, s_acc):
    c = pl.program_id(0)
    j = pl.program_id(1)
    jj = c * NVC + j

    @pl.when(j == 0)
    def _():
        x4 = x4_ref[...]
        xf = x4 * jax.lax.rsqrt(jnp.mean(x4 * x4, axis=1, keepdims=True)
                                + 1e-5) * nfw_ref[...]
        xf_s[...] = xf
        tdot_ref[...] = jnp.sum(xf * etgt_ref[...], axis=1, keepdims=True)
        s_acc[...] = jnp.zeros_like(s_acc)

    xf = xf_s[...]
    blk = jnp.einsum("ld,vd->lv", xf, e_ref[...],
                     preferred_element_type=jnp.float32)
    out_ref[...] = blk

    @pl.when(jj < NV - 1)
    def _():
        s_acc[...] = s_acc[...] + jnp.sum(jnp.exp(blk), axis=1, keepdims=True)

    @pl.when(jj >= NV - 1)
    def _():
        cols = jj * BV + jax.lax.broadcasted_iota(jnp.int32, (1, BV), 1)
        blkm = jnp.where(cols < VOCAB, blk, -jnp.inf)
        s_acc[...] = s_acc[...] + jnp.sum(jnp.exp(blkm), axis=1, keepdims=True)

    @pl.when(j == NVC - 1)
    def _():
        s_ref[...] = jnp.broadcast_to(s_acc[...], s_ref.shape)


def _head_call(x4, etgt, nfw, emb):
    full = lambda shape: pl.BlockSpec(shape, lambda c, j: tuple(0 for _ in shape))
    vblk = lambda c, j: (jnp.minimum(c * NVC + j, NV - 1), 0)
    return pl.pallas_call(
        _head_kernel,
        out_shape=[
            jax.ShapeDtypeStruct((L, VOCAB), jnp.float32),
            jax.ShapeDtypeStruct((L, 256), jnp.float32),
            jax.ShapeDtypeStruct((L, 1), jnp.float32),
        ],
        grid=(2, NVC),
        in_specs=[
            full((L, D_MODEL)),
            full((L, D_MODEL)),
            full((1, D_MODEL)),
            pl.BlockSpec((BV, D_MODEL), vblk),
        ],
        out_specs=[
            pl.BlockSpec((L, BV), lambda c, j: (0, jnp.minimum(c * NVC + j, NV - 1))),
            pl.BlockSpec((L, 128), lambda c, j: (0, c)),
            full((L, 1)),
        ],
        scratch_shapes=[
            pltpu.VMEM((L, D_MODEL), jnp.float32),
            pltpu.VMEM((L, 1), jnp.float32),
        ],
        compiler_params=pltpu.CompilerParams(
            dimension_semantics=("parallel", "arbitrary"),
            vmem_limit_bytes=60 * 1024 * 1024,
        ),
        name="lm_head",
    )(x4, etgt, nfw, emb)


def kernel(idx, targets, embedding, norm_w, in_proj_w, conv_w, conv_b,
           x_proj_w, dt_proj_w, dt_proj_b, A_log, D_param, out_proj_w,
           norm_f_w):
    tclip = jnp.clip(targets[0], 0, VOCAB - 1)
    cidx = jnp.concatenate([idx[0], tclip])
    gathered = _gather_call(cidx, embedding)
    x = gathered[:L]
    etgt = gathered[L:]

    for l in range(N_LAYER):
        xpw = jnp.pad(x_proj_w[l], ((0, 128 - (DT_RANK + 2 * D_STATE)), (0, 0)))
        dtw = jnp.pad(dt_proj_w[l], ((0, 0), (0, 128 - DT_RANK)))
        x = _layer_call(
            x,
            in_proj_w[l],
            conv_w[l].T,
            conv_b[l][None, :],
            xpw,
            dtw,
            dt_proj_b[l][None, :],
            A_log[l].T,
            D_param[l][None, :],
            out_proj_w[l],
            norm_w[l][None, :],
        )

    logits2d, s_out, tdot = _head_call(x, etgt, norm_f_w[None, :], embedding)
    logits = logits2d.reshape(1, L, VOCAB)

    lse = jnp.log(s_out[:, 0] + s_out[:, 128])
    maskv = (targets[0] >= 0).astype(jnp.float32)
    nll = (lse - tdot[:, 0]) * maskv
    loss = jnp.sum(nll) / jnp.maximum(jnp.sum(maskv), 1.0)
    return (logits, loss)
